# baseline (speedup 1.0000x reference)
import functools

import jax
import jax.numpy as jnp
import numpy as np
from jax import lax
from jax.experimental import pallas as pl
from jax.experimental.pallas import tpu as pltpu
from jax.experimental.pallas import tpu_sc as plsc

_POS_INDICES = [120, 124, 128, 132, 136, 140, 144, 148, 152, 237, 241, 245,
                249, 253, 257, 261, 265, 269, 354, 358, 362, 366, 370, 374,
                378, 382, 386, 471, 475, 479, 483, 487, 491, 495, 499, 503,
                588, 592, 596, 600, 604, 608, 612, 616, 620, 705, 709, 713,
                717, 721, 725, 729, 733, 737, 822, 826, 830, 834, 838, 842,
                846, 850, 854, 48, 53, 60, 67, 73, 157, 352, 388, 583, 586,
                817, 901, 906, 913, 920, 926]


def _positions():
    width, height = 39, 25
    pf = np.zeros((79, 6), dtype=np.float32)
    for i, p in enumerate(_POS_INDICES):
        x = p % width
        y = p // width
        xn = x / (width - 1)
        yn = y / (height - 1)
        pf[i, 0] = xn
        pf[i, 1] = yn
        pf[i, 2] = (np.sin(xn * 2 * np.pi) + 1) / 2
        pf[i, 3] = (np.cos(xn * 2 * np.pi) + 1) / 2
        pf[i, 4] = (np.sin(yn * 2 * np.pi) + 1) / 2
        pf[i, 5] = (np.cos(yn * 2 * np.pi) + 1) / 2
    return pf


_POS = _positions()

NP = 80
NA = 128
TCOL = 96
G1 = 8
G2 = 8
_INV_SQRT_D = 1.0 / 16.0


def _proj_body(x_ref, w_ref, b_ref, we_ref, vs_ref, sc_ref):
    x = x_ref[...]
    y = jnp.dot(x, w_ref[...], preferred_element_type=jnp.float32) + b_ref[...]
    q = y[:, :256]
    k = y[:, 256:512]
    vs_ref[...] = y[:, 512:]
    w = we_ref[...]
    t = jnp.sum(q * w, axis=1, keepdims=True) * _INV_SQRT_D
    zpad = jnp.zeros((NA - NP, 256), jnp.float32)
    colid = lax.broadcasted_iota(jnp.int32, (NP, NA), 1)
    for i in range(G1):
        qi = q[NP * i:NP * i + NP]
        ki = k[NP * i:NP * i + NP]
        kaug = jnp.concatenate([ki, zpad], axis=0)
        s = lax.dot_general(qi, kaug, (((1,), (1,)), ((), ())),
                            preferred_element_type=jnp.float32) * _INV_SQRT_D
        ti = t[NP * i:NP * i + NP]
        c = jnp.max(s, axis=1, keepdims=True) + jnp.abs(ti)
        sc_ref[NP * i:NP * i + NP, :] = jnp.where(colid == TCOL, ti, s - c)


def _sc_edge_body(sc_hbm, pidx_hbm, a_hbm, zero_hbm, out_hbm,
                  pidx_v, a_v, sc_v0, sc_v1, acc_v0, acc_v1,
                  sem_sc0, sem_sc1, sem_z0, sem_z1, sem_o0, sem_o1,
                  n_graphs, n_groups):
    cid = lax.axis_index("c")
    sid = lax.axis_index("s")
    wid = sid * 2 + cid
    gpw = n_graphs // 32
    pltpu.sync_copy(pidx_hbm, pidx_v)
    pltpu.sync_copy(a_hbm, a_v)
    lane80 = 80 + lax.iota(jnp.int32, 16)
    tcol = jnp.full((16,), TCOL, jnp.int32)

    bufs = [(sc_v0, acc_v0, sem_sc0, sem_z0, sem_o0),
            (sc_v1, acc_v1, sem_sc1, sem_z1, sem_o1)]

    def make_edge(sc_v, acc_v):
        def edge_body(j, _):
            slc = pl.ds(j * 16, 16)
            pi = pidx_v[slc]
            ai = a_v[slc]
            di = lax.shift_right_logical(pi, 7)
            si = jnp.bitwise_and(pi, NA - 1)
            base = plsc.load_gather(sc_v, [di, si])
            tv = plsc.load_gather(sc_v, [di, tcol])
            e = jnp.exp(base + ai * tv)
            plsc.addupdate_scatter(acc_v, [di, si], e)
            plsc.addupdate_scatter(acc_v, [di, lane80], e * ai)
            return 0
        return edge_body

    g0 = wid * gpw
    in_h = {}
    z_h = {}
    out_h = {}
    in_h[0] = pltpu.async_copy(sc_hbm.at[pl.ds(g0 * NP, NP)],
                               bufs[0][0], bufs[0][2])
    z_h[0] = pltpu.async_copy(zero_hbm, bufs[0][1], bufs[0][3])
    for gi in range(gpw):
        sc_v, acc_v, s_sc, s_z, s_o = bufs[gi % 2]
        in_h.pop(gi).wait()
        z_h.pop(gi).wait()
        if gi + 1 < gpw:
            nb = bufs[(gi + 1) % 2]
            if gi >= 1:
                out_h.pop(gi - 1).wait()
            in_h[gi + 1] = pltpu.async_copy(
                sc_hbm.at[pl.ds((g0 + gi + 1) * NP, NP)], nb[0], nb[2])
            z_h[gi + 1] = pltpu.async_copy(zero_hbm, nb[1], nb[3])
        lax.fori_loop(0, n_groups, make_edge(sc_v, acc_v), 0, unroll=25)
        out_h[gi] = pltpu.async_copy(
            acc_v, out_hbm.at[pl.ds((g0 + gi) * NP, NP)], s_o)
    for h in out_h.values():
        h.wait()


def _sc_edge(scp, pidx, ap, zero):
    n = scp.shape[0]
    e = pidx.shape[0]
    mesh = plsc.VectorSubcoreMesh(core_axis_name="c", subcore_axis_name="s")
    fn = pl.kernel(
        functools.partial(_sc_edge_body, n_graphs=n // NP, n_groups=e // 16),
        mesh=mesh,
        compiler_params=pltpu.CompilerParams(needs_layout_passes=False),
        out_type=jax.ShapeDtypeStruct((n, NA), jnp.float32),
        scratch_types=[
            pltpu.VMEM((e,), jnp.int32),
            pltpu.VMEM((e,), jnp.float32),
            pltpu.VMEM((NP, NA), jnp.float32),
            pltpu.VMEM((NP, NA), jnp.float32),
            pltpu.VMEM((NP, NA), jnp.float32),
            pltpu.VMEM((NP, NA), jnp.float32),
            pltpu.SemaphoreType.DMA,
            pltpu.SemaphoreType.DMA,
            pltpu.SemaphoreType.DMA,
            pltpu.SemaphoreType.DMA,
            pltpu.SemaphoreType.DMA,
            pltpu.SemaphoreType.DMA,
        ],
    )
    return fn(scp, pidx, ap, zero)


def _layernorm(x, g, b):
    mu = jnp.mean(x, axis=1, keepdims=True)
    xc = x - mu
    var = jnp.mean(xc * xc, axis=1, keepdims=True)
    return xc * lax.rsqrt(var + 1e-5) * g + b


def _lrelu(x):
    return jnp.where(x >= 0, x, 0.01 * x)


def _head_body(a_ref, vs_ref, x_ref, we_ref, w1a_ref, w1c_ref, w1b_ref,
               b1_ref, g1_ref, be1_ref, w2_ref, b2_ref, g2_ref, be2_ref,
               w3_ref, b3_ref, act_ref, reg_ref, *, n_real, total_nodes,
               n_steps):
    step = pl.program_id(0)
    rows = NP * G2
    amat = a_ref[...]
    vs = vs_ref[...]
    x = x_ref[...]
    w = we_ref[...]
    wpad = jnp.broadcast_to(w, (NA - NP, 256))
    colA = lax.broadcasted_iota(jnp.int32, (rows, NA), 1)
    colX = lax.broadcasted_iota(jnp.int32, (rows, 128), 1)
    msgs = []
    for i in range(G2):
        ai = amat[NP * i:NP * i + NP]
        vi = vs[NP * i:NP * i + NP, :256]
        vaug = jnp.concatenate([vi, wpad], axis=0)
        msgs.append(jnp.dot(ai, vaug, preferred_element_type=jnp.float32))
    msg = jnp.concatenate(msgs, axis=0)
    den = jnp.sum(jnp.where(colA < NP, amat, 0.0), axis=1, keepdims=True)
    out = msg / (den + 1e-16) + vs[:, 256:]
    h = jnp.maximum(out, 0.0)
    xc1 = jnp.where(colX == 1, x, 0.0)
    ta_cols = []
    for i in range(G2):
        ta_i = jnp.sum(xc1[NP * i:NP * i + NP])
        ta_cols.append(jnp.full((NP, 1), ta_i, jnp.float32))
    ta = jnp.concatenate(ta_cols, axis=0)
    o1 = (jnp.dot(h, w1a_ref[...], preferred_element_type=jnp.float32)
          + jnp.dot(x, w1c_ref[...], preferred_element_type=jnp.float32)
          + ta * w1b_ref[...] + b1_ref[...])
    x1 = _lrelu(_layernorm(o1, g1_ref[...], be1_ref[...]))
    o2 = jnp.dot(x1, w2_ref[...],
                 preferred_element_type=jnp.float32) + b2_ref[...]
    x2 = _lrelu(_layernorm(o2, g2_ref[...], be2_ref[...]))
    z = jnp.sum(x2 * w3_ref[...], axis=1, keepdims=True) + b3_ref[0, 0]
    conc = jnp.maximum(z, 0.0) + jnp.log(1.0 + jnp.exp(-jnp.abs(z)))
    rowm = (lax.broadcasted_iota(jnp.int32, (rows, 1), 0) % NP) < n_real
    concm = jnp.where(rowm, conc, 0.0)
    regs = jnp.sum(jnp.abs(concm))
    for i in range(G2):
        ci = concm[NP * i:NP * i + NP]
        ssum = jnp.sum(ci)
        act_ref[NP * i:NP * i + NP, :] = ci / (ssum + 1e-20)
    tot = jnp.where(step == 0, 0.0, reg_ref[...]) + regs
    reg_ref[...] = jnp.where(step == n_steps - 1, tot / total_nodes, tot)


def _proj_call(xf, wall, ball, we):
    n = xf.shape[0]
    rows = NP * G1
    grid = (n // rows,)
    return pl.pallas_call(
        _proj_body,
        grid=grid,
        in_specs=[
            pl.BlockSpec((rows, 128), lambda i: (i, 0)),
            pl.BlockSpec((128, 1024), lambda i: (0, 0)),
            pl.BlockSpec((1, 1024), lambda i: (0, 0)),
            pl.BlockSpec((1, 256), lambda i: (0, 0)),
        ],
        out_specs=[
            pl.BlockSpec((rows, 512), lambda i: (i, 0)),
            pl.BlockSpec((rows, NA), lambda i: (i, 0)),
        ],
        out_shape=[
            jax.ShapeDtypeStruct((n, 512), jnp.float32),
            jax.ShapeDtypeStruct((n, NA), jnp.float32),
        ],
    )(xf, wall, ball, we)


def _head_call(af, vs, xf, we, w1a, w1c, w1b, b1, g1, be1, w2, b2, g2, be2,
               w3r, b3, n_real, total_nodes):
    n = af.shape[0]
    rows = NP * G2
    grid = (n // rows,)
    body = functools.partial(_head_body, n_real=n_real,
                             total_nodes=total_nodes, n_steps=n // rows)
    const = lambda i: (0, 0)
    return pl.pallas_call(
        body,
        grid=grid,
        in_specs=[
            pl.BlockSpec((rows, NA), lambda i: (i, 0)),
            pl.BlockSpec((rows, 512), lambda i: (i, 0)),
            pl.BlockSpec((rows, 128), lambda i: (i, 0)),
            pl.BlockSpec((1, 256), const),
            pl.BlockSpec((256, 256), const),
            pl.BlockSpec((128, 256), const),
            pl.BlockSpec((1, 256), const),
            pl.BlockSpec((1, 256), const),
            pl.BlockSpec((1, 256), const),
            pl.BlockSpec((1, 256), const),
            pl.BlockSpec((256, 256), const),
            pl.BlockSpec((1, 256), const),
            pl.BlockSpec((1, 256), const),
            pl.BlockSpec((1, 256), const),
            pl.BlockSpec((1, 256), const),
            pl.BlockSpec((1, 1), const),
        ],
        out_specs=[
            pl.BlockSpec((rows, 1), lambda i: (i, 0)),
            pl.BlockSpec((1, 1), const),
        ],
        out_shape=[
            jax.ShapeDtypeStruct((n, 1), jnp.float32),
            jax.ShapeDtypeStruct((1, 1), jnp.float32),
        ],
    )(af, vs, xf, we, w1a, w1c, w1b, b1, g1, be1, w2, b2, g2, be2, w3r, b3)


def kernel(state, edge_index, edge_attr, Wq, bq, Wk, bk, Wv, bv, We, Wskip,
           bskip, W1, b1, g1, beta1, W2, b2, g2, beta2, W3, b3):
    B, Npg, Fin = state.shape
    x = jnp.concatenate(
        [state, jnp.broadcast_to(_POS[None], (B, Npg, 6))], axis=-1)
    xf = jnp.pad(x, ((0, 0), (0, NP - Npg), (0, 0))).reshape(B * NP, -1)

    wall = jnp.concatenate([Wq, Wk, Wv, Wskip], axis=1)
    ball = jnp.concatenate([bq, bk, bv, bskip])[None]
    vs, scp = _proj_call(xf, wall, ball, We)

    src = edge_index[0].astype(jnp.int32)
    dst = edge_index[1].astype(jnp.int32)
    pidx = dst * NA + src
    ap = edge_attr[:, 0]
    zero = jnp.zeros((NP, NA), jnp.float32)

    amat = _sc_edge(scp, pidx, ap, zero)

    act, reg = _head_call(
        amat, vs, xf, We,
        W1[:256], W1[257:], W1[256:257], b1[None], g1[None], beta1[None],
        W2, b2[None], g2[None], beta2[None], W3.T, b3[None],
        Npg, B * Npg)
    action = act.reshape(B, NP)[:, :Npg]
    return (action, reg[0, 0])


# speedup vs baseline: 1.0290x; 1.0290x over previous
import functools

import jax
import jax.numpy as jnp
import numpy as np
from jax import lax
from jax.experimental import pallas as pl
from jax.experimental.pallas import tpu as pltpu
from jax.experimental.pallas import tpu_sc as plsc

_POS_INDICES = [120, 124, 128, 132, 136, 140, 144, 148, 152, 237, 241, 245,
                249, 253, 257, 261, 265, 269, 354, 358, 362, 366, 370, 374,
                378, 382, 386, 471, 475, 479, 483, 487, 491, 495, 499, 503,
                588, 592, 596, 600, 604, 608, 612, 616, 620, 705, 709, 713,
                717, 721, 725, 729, 733, 737, 822, 826, 830, 834, 838, 842,
                846, 850, 854, 48, 53, 60, 67, 73, 157, 352, 388, 583, 586,
                817, 901, 906, 913, 920, 926]


def _positions():
    width, height = 39, 25
    pf = np.zeros((79, 6), dtype=np.float32)
    for i, p in enumerate(_POS_INDICES):
        x = p % width
        y = p // width
        xn = x / (width - 1)
        yn = y / (height - 1)
        pf[i, 0] = xn
        pf[i, 1] = yn
        pf[i, 2] = (np.sin(xn * 2 * np.pi) + 1) / 2
        pf[i, 3] = (np.cos(xn * 2 * np.pi) + 1) / 2
        pf[i, 4] = (np.sin(yn * 2 * np.pi) + 1) / 2
        pf[i, 5] = (np.cos(yn * 2 * np.pi) + 1) / 2
    return pf


_POS = _positions()

NP = 80
NA = 128
TCOL = 96
G1 = 8
G2 = 8
_INV_SQRT_D = 1.0 / 16.0


def _proj_body(x_ref, w_ref, b_ref, we_ref, vs_ref, sc_ref):
    x = x_ref[...]
    y = jnp.dot(x, w_ref[...], preferred_element_type=jnp.float32) + b_ref[...]
    q = y[:, :256]
    k = y[:, 256:512]
    vs_ref[...] = y[:, 512:]
    w = we_ref[...]
    t = jnp.sum(q * w, axis=1, keepdims=True) * _INV_SQRT_D
    zpad = jnp.zeros((NA - NP, 256), jnp.float32)
    colid = lax.broadcasted_iota(jnp.int32, (NP, NA), 1)
    for i in range(G1):
        qi = q[NP * i:NP * i + NP]
        ki = k[NP * i:NP * i + NP]
        kaug = jnp.concatenate([ki, zpad], axis=0)
        s = lax.dot_general(qi, kaug, (((1,), (1,)), ((), ())),
                            preferred_element_type=jnp.float32) * _INV_SQRT_D
        ti = t[NP * i:NP * i + NP]
        c = jnp.max(s, axis=1, keepdims=True) + jnp.abs(ti)
        sc_ref[NP * i:NP * i + NP, :] = jnp.where(colid == TCOL, ti, s - c)


def _sc_edge_body(sc_hbm, pidx_hbm, a_hbm, zero_hbm, out_hbm,
                  pidx_v, a_v, sc_v0, sc_v1, acc_v0, acc_v1,
                  sem_sc0, sem_sc1, sem_z0, sem_z1, sem_o0, sem_o1,
                  n_graphs, n_groups):
    cid = lax.axis_index("c")
    sid = lax.axis_index("s")
    wid = sid * 2 + cid
    gpw = n_graphs // 32
    pltpu.sync_copy(pidx_hbm, pidx_v)
    pltpu.sync_copy(a_hbm, a_v)
    lane80 = 80 + lax.iota(jnp.int32, 16)
    tcol = jnp.full((16,), TCOL, jnp.int32)

    bufs = [(sc_v0, acc_v0, sem_sc0, sem_z0, sem_o0),
            (sc_v1, acc_v1, sem_sc1, sem_z1, sem_o1)]

    def make_edge(sc_v, acc_v):
        def edge_body(j):
            slc = pl.ds(j * 16, 16)
            pi = pidx_v[slc]
            ai = a_v[slc]
            di = lax.shift_right_logical(pi, 7)
            si = jnp.bitwise_and(pi, NA - 1)
            base = plsc.load_gather(sc_v, [di, si])
            tv = plsc.load_gather(sc_v, [di, tcol])
            e = jnp.exp(base + ai * tv)
            plsc.addupdate_scatter(acc_v, [di, si], e)
            plsc.addupdate_scatter(acc_v, [di, lane80], e * ai)
        return edge_body

    g0 = wid * gpw
    in_h = {}
    z_h = {}
    out_h = {}
    in_h[0] = pltpu.async_copy(sc_hbm.at[pl.ds(g0 * NP, NP)],
                               bufs[0][0], bufs[0][2])
    z_h[0] = pltpu.async_copy(zero_hbm, bufs[0][1], bufs[0][3])
    for gi in range(gpw):
        sc_v, acc_v, s_sc, s_z, s_o = bufs[gi % 2]
        in_h.pop(gi).wait()
        z_h.pop(gi).wait()
        if gi + 1 < gpw:
            nb = bufs[(gi + 1) % 2]
            if gi >= 1:
                out_h.pop(gi - 1).wait()
            in_h[gi + 1] = pltpu.async_copy(
                sc_hbm.at[pl.ds((g0 + gi + 1) * NP, NP)], nb[0], nb[2])
            z_h[gi + 1] = pltpu.async_copy(zero_hbm, nb[1], nb[3])
        plsc.parallel_loop(0, n_groups, unroll=8)(make_edge(sc_v, acc_v))
        out_h[gi] = pltpu.async_copy(
            acc_v, out_hbm.at[pl.ds((g0 + gi) * NP, NP)], s_o)
    for h in out_h.values():
        h.wait()


def _sc_edge(scp, pidx, ap, zero):
    n = scp.shape[0]
    e = pidx.shape[0]
    mesh = plsc.VectorSubcoreMesh(core_axis_name="c", subcore_axis_name="s")
    fn = pl.kernel(
        functools.partial(_sc_edge_body, n_graphs=n // NP, n_groups=e // 16),
        mesh=mesh,
        compiler_params=pltpu.CompilerParams(needs_layout_passes=False),
        out_type=jax.ShapeDtypeStruct((n, NA), jnp.float32),
        scratch_types=[
            pltpu.VMEM((e,), jnp.int32),
            pltpu.VMEM((e,), jnp.float32),
            pltpu.VMEM((NP, NA), jnp.float32),
            pltpu.VMEM((NP, NA), jnp.float32),
            pltpu.VMEM((NP, NA), jnp.float32),
            pltpu.VMEM((NP, NA), jnp.float32),
            pltpu.SemaphoreType.DMA,
            pltpu.SemaphoreType.DMA,
            pltpu.SemaphoreType.DMA,
            pltpu.SemaphoreType.DMA,
            pltpu.SemaphoreType.DMA,
            pltpu.SemaphoreType.DMA,
        ],
    )
    return fn(scp, pidx, ap, zero)


def _layernorm(x, g, b):
    mu = jnp.mean(x, axis=1, keepdims=True)
    xc = x - mu
    var = jnp.mean(xc * xc, axis=1, keepdims=True)
    return xc * lax.rsqrt(var + 1e-5) * g + b


def _lrelu(x):
    return jnp.where(x >= 0, x, 0.01 * x)


def _head_body(a_ref, vs_ref, x_ref, we_ref, w1a_ref, w1c_ref, w1b_ref,
               b1_ref, g1_ref, be1_ref, w2_ref, b2_ref, g2_ref, be2_ref,
               w3_ref, b3_ref, act_ref, reg_ref, *, n_real, total_nodes,
               n_steps):
    step = pl.program_id(0)
    rows = NP * G2
    amat = a_ref[...]
    vs = vs_ref[...]
    x = x_ref[...]
    w = we_ref[...]
    wpad = jnp.broadcast_to(w, (NA - NP, 256))
    colA = lax.broadcasted_iota(jnp.int32, (rows, NA), 1)
    colX = lax.broadcasted_iota(jnp.int32, (rows, 128), 1)
    msgs = []
    for i in range(G2):
        ai = amat[NP * i:NP * i + NP]
        vi = vs[NP * i:NP * i + NP, :256]
        vaug = jnp.concatenate([vi, wpad], axis=0)
        msgs.append(jnp.dot(ai, vaug, preferred_element_type=jnp.float32))
    msg = jnp.concatenate(msgs, axis=0)
    den = jnp.sum(jnp.where(colA < NP, amat, 0.0), axis=1, keepdims=True)
    out = msg / (den + 1e-16) + vs[:, 256:]
    h = jnp.maximum(out, 0.0)
    xc1 = jnp.where(colX == 1, x, 0.0)
    ta_cols = []
    for i in range(G2):
        ta_i = jnp.sum(xc1[NP * i:NP * i + NP])
        ta_cols.append(jnp.full((NP, 1), ta_i, jnp.float32))
    ta = jnp.concatenate(ta_cols, axis=0)
    o1 = (jnp.dot(h, w1a_ref[...], preferred_element_type=jnp.float32)
          + jnp.dot(x, w1c_ref[...], preferred_element_type=jnp.float32)
          + ta * w1b_ref[...] + b1_ref[...])
    x1 = _lrelu(_layernorm(o1, g1_ref[...], be1_ref[...]))
    o2 = jnp.dot(x1, w2_ref[...],
                 preferred_element_type=jnp.float32) + b2_ref[...]
    x2 = _lrelu(_layernorm(o2, g2_ref[...], be2_ref[...]))
    z = jnp.sum(x2 * w3_ref[...], axis=1, keepdims=True) + b3_ref[0, 0]
    conc = jnp.maximum(z, 0.0) + jnp.log(1.0 + jnp.exp(-jnp.abs(z)))
    rowm = (lax.broadcasted_iota(jnp.int32, (rows, 1), 0) % NP) < n_real
    concm = jnp.where(rowm, conc, 0.0)
    regs = jnp.sum(jnp.abs(concm))
    for i in range(G2):
        ci = concm[NP * i:NP * i + NP]
        ssum = jnp.sum(ci)
        act_ref[NP * i:NP * i + NP, :] = ci / (ssum + 1e-20)
    tot = jnp.where(step == 0, 0.0, reg_ref[...]) + regs
    reg_ref[...] = jnp.where(step == n_steps - 1, tot / total_nodes, tot)


def _proj_call(xf, wall, ball, we):
    n = xf.shape[0]
    rows = NP * G1
    grid = (n // rows,)
    return pl.pallas_call(
        _proj_body,
        grid=grid,
        in_specs=[
            pl.BlockSpec((rows, 128), lambda i: (i, 0)),
            pl.BlockSpec((128, 1024), lambda i: (0, 0)),
            pl.BlockSpec((1, 1024), lambda i: (0, 0)),
            pl.BlockSpec((1, 256), lambda i: (0, 0)),
        ],
        out_specs=[
            pl.BlockSpec((rows, 512), lambda i: (i, 0)),
            pl.BlockSpec((rows, NA), lambda i: (i, 0)),
        ],
        out_shape=[
            jax.ShapeDtypeStruct((n, 512), jnp.float32),
            jax.ShapeDtypeStruct((n, NA), jnp.float32),
        ],
    )(xf, wall, ball, we)


def _head_call(af, vs, xf, we, w1a, w1c, w1b, b1, g1, be1, w2, b2, g2, be2,
               w3r, b3, n_real, total_nodes):
    n = af.shape[0]
    rows = NP * G2
    grid = (n // rows,)
    body = functools.partial(_head_body, n_real=n_real,
                             total_nodes=total_nodes, n_steps=n // rows)
    const = lambda i: (0, 0)
    return pl.pallas_call(
        body,
        grid=grid,
        in_specs=[
            pl.BlockSpec((rows, NA), lambda i: (i, 0)),
            pl.BlockSpec((rows, 512), lambda i: (i, 0)),
            pl.BlockSpec((rows, 128), lambda i: (i, 0)),
            pl.BlockSpec((1, 256), const),
            pl.BlockSpec((256, 256), const),
            pl.BlockSpec((128, 256), const),
            pl.BlockSpec((1, 256), const),
            pl.BlockSpec((1, 256), const),
            pl.BlockSpec((1, 256), const),
            pl.BlockSpec((1, 256), const),
            pl.BlockSpec((256, 256), const),
            pl.BlockSpec((1, 256), const),
            pl.BlockSpec((1, 256), const),
            pl.BlockSpec((1, 256), const),
            pl.BlockSpec((1, 256), const),
            pl.BlockSpec((1, 1), const),
        ],
        out_specs=[
            pl.BlockSpec((rows, 1), lambda i: (i, 0)),
            pl.BlockSpec((1, 1), const),
        ],
        out_shape=[
            jax.ShapeDtypeStruct((n, 1), jnp.float32),
            jax.ShapeDtypeStruct((1, 1), jnp.float32),
        ],
    )(af, vs, xf, we, w1a, w1c, w1b, b1, g1, be1, w2, b2, g2, be2, w3r, b3)


def kernel(state, edge_index, edge_attr, Wq, bq, Wk, bk, Wv, bv, We, Wskip,
           bskip, W1, b1, g1, beta1, W2, b2, g2, beta2, W3, b3):
    B, Npg, Fin = state.shape
    x = jnp.concatenate(
        [state, jnp.broadcast_to(_POS[None], (B, Npg, 6))], axis=-1)
    xf = jnp.pad(x, ((0, 0), (0, NP - Npg), (0, 0))).reshape(B * NP, -1)

    wall = jnp.concatenate([Wq, Wk, Wv, Wskip], axis=1)
    ball = jnp.concatenate([bq, bk, bv, bskip])[None]
    vs, scp = _proj_call(xf, wall, ball, We)

    src = edge_index[0].astype(jnp.int32)
    dst = edge_index[1].astype(jnp.int32)
    pidx = dst * NA + src
    ap = edge_attr[:, 0]
    zero = jnp.zeros((NP, NA), jnp.float32)

    amat = _sc_edge(scp, pidx, ap, zero)

    act, reg = _head_call(
        amat, vs, xf, We,
        W1[:256], W1[257:], W1[256:257], b1[None], g1[None], beta1[None],
        W2, b2[None], g2[None], beta2[None], W3.T, b3[None],
        Npg, B * Npg)
    action = act.reshape(B, NP)[:, :Npg]
    return (action, reg[0, 0])


# speedup vs baseline: 1.0474x; 1.0179x over previous
import functools

import jax
import jax.numpy as jnp
import numpy as np
from jax import lax
from jax.experimental import pallas as pl
from jax.experimental.pallas import tpu as pltpu
from jax.experimental.pallas import tpu_sc as plsc

_POS_INDICES = [120, 124, 128, 132, 136, 140, 144, 148, 152, 237, 241, 245,
                249, 253, 257, 261, 265, 269, 354, 358, 362, 366, 370, 374,
                378, 382, 386, 471, 475, 479, 483, 487, 491, 495, 499, 503,
                588, 592, 596, 600, 604, 608, 612, 616, 620, 705, 709, 713,
                717, 721, 725, 729, 733, 737, 822, 826, 830, 834, 838, 842,
                846, 850, 854, 48, 53, 60, 67, 73, 157, 352, 388, 583, 586,
                817, 901, 906, 913, 920, 926]


def _positions():
    width, height = 39, 25
    pf = np.zeros((79, 6), dtype=np.float32)
    for i, p in enumerate(_POS_INDICES):
        x = p % width
        y = p // width
        xn = x / (width - 1)
        yn = y / (height - 1)
        pf[i, 0] = xn
        pf[i, 1] = yn
        pf[i, 2] = (np.sin(xn * 2 * np.pi) + 1) / 2
        pf[i, 3] = (np.cos(xn * 2 * np.pi) + 1) / 2
        pf[i, 4] = (np.sin(yn * 2 * np.pi) + 1) / 2
        pf[i, 5] = (np.cos(yn * 2 * np.pi) + 1) / 2
    return pf


_POS = _positions()

NP = 80
NA = 128
TCOL = 96
G1 = 8
G2 = 8
_INV_SQRT_D = 1.0 / 16.0


def _proj_body(x_ref, w_ref, b_ref, we_ref, vs_ref, sc_ref):
    x = x_ref[...]
    y = jnp.dot(x, w_ref[...], preferred_element_type=jnp.float32) + b_ref[...]
    q = y[:, :256]
    k = y[:, 256:512]
    vs_ref[...] = y[:, 512:].astype(jnp.bfloat16)
    w = we_ref[...]
    t = jnp.sum(q * w, axis=1, keepdims=True) * _INV_SQRT_D
    zpad = jnp.zeros((NA - NP, 256), jnp.float32)
    colid = lax.broadcasted_iota(jnp.int32, (NP, NA), 1)
    for i in range(G1):
        qi = q[NP * i:NP * i + NP]
        ki = k[NP * i:NP * i + NP]
        kaug = jnp.concatenate([ki, zpad], axis=0)
        s = lax.dot_general(qi, kaug, (((1,), (1,)), ((), ())),
                            preferred_element_type=jnp.float32) * _INV_SQRT_D
        ti = t[NP * i:NP * i + NP]
        c = jnp.max(s, axis=1, keepdims=True) + jnp.abs(ti)
        sc_ref[NP * i:NP * i + NP, :] = jnp.where(colid == TCOL, ti, s - c)


def _sc_edge_body(sc_hbm, ei_hbm, ea_hbm, zero_hbm, out_hbm,
                  pidx_v, dst_v, a_v, sc_v0, sc_v1, acc_v0, acc_v1,
                  sem_sc0, sem_sc1, sem_z0, sem_z1, sem_o0, sem_o1,
                  n_graphs, n_groups):
    cid = lax.axis_index("c")
    sid = lax.axis_index("s")
    wid = sid * 2 + cid
    gpw = n_graphs // 32
    pltpu.sync_copy(ei_hbm.at[0], pidx_v)
    pltpu.sync_copy(ei_hbm.at[1], dst_v)
    pltpu.sync_copy(ea_hbm, a_v)
    lane80 = 80 + lax.iota(jnp.int32, 16)
    tcol = jnp.full((16,), TCOL, jnp.int32)

    def pack_body(j):
        slc = pl.ds(j * 16, 16)
        pidx_v[slc] = dst_v[slc] * NA + pidx_v[slc]

    plsc.parallel_loop(0, n_groups, unroll=8)(pack_body)

    bufs = [(sc_v0, acc_v0, sem_sc0, sem_z0, sem_o0),
            (sc_v1, acc_v1, sem_sc1, sem_z1, sem_o1)]

    def make_edge(sc_v, acc_v):
        def edge_body(j):
            slc = pl.ds(j * 16, 16)
            pi = pidx_v[slc]
            ai = a_v[slc]
            di = lax.shift_right_logical(pi, 7)
            si = jnp.bitwise_and(pi, NA - 1)
            base = plsc.load_gather(sc_v, [di, si])
            tv = plsc.load_gather(sc_v, [di, tcol])
            e = jnp.exp(base + ai * tv)
            plsc.addupdate_scatter(acc_v, [di, si], e)
            plsc.addupdate_scatter(acc_v, [di, lane80], e * ai)
        return edge_body

    g0 = wid * gpw
    in_h = {}
    z_h = {}
    out_h = {}
    in_h[0] = pltpu.async_copy(sc_hbm.at[pl.ds(g0 * NP, NP)],
                               bufs[0][0], bufs[0][2])
    z_h[0] = pltpu.async_copy(zero_hbm, bufs[0][1], bufs[0][3])
    for gi in range(gpw):
        sc_v, acc_v, s_sc, s_z, s_o = bufs[gi % 2]
        in_h.pop(gi).wait()
        z_h.pop(gi).wait()
        if gi + 1 < gpw:
            nb = bufs[(gi + 1) % 2]
            if gi >= 1:
                out_h.pop(gi - 1).wait()
            in_h[gi + 1] = pltpu.async_copy(
                sc_hbm.at[pl.ds((g0 + gi + 1) * NP, NP)], nb[0], nb[2])
            z_h[gi + 1] = pltpu.async_copy(zero_hbm, nb[1], nb[3])
        plsc.parallel_loop(0, n_groups, unroll=8)(make_edge(sc_v, acc_v))
        out_h[gi] = pltpu.async_copy(
            acc_v, out_hbm.at[pl.ds((g0 + gi) * NP, NP)], s_o)
    for h in out_h.values():
        h.wait()


def _sc_edge(scp, ei, ea, zero):
    n = scp.shape[0]
    e = ei.shape[1]
    mesh = plsc.VectorSubcoreMesh(core_axis_name="c", subcore_axis_name="s")
    fn = pl.kernel(
        functools.partial(_sc_edge_body, n_graphs=n // NP, n_groups=e // 16),
        mesh=mesh,
        compiler_params=pltpu.CompilerParams(needs_layout_passes=False),
        out_type=jax.ShapeDtypeStruct((n, NA), jnp.float32),
        scratch_types=[
            pltpu.VMEM((e,), jnp.int32),
            pltpu.VMEM((e,), jnp.int32),
            pltpu.VMEM((e,), jnp.float32),
            pltpu.VMEM((NP, NA), jnp.float32),
            pltpu.VMEM((NP, NA), jnp.float32),
            pltpu.VMEM((NP, NA), jnp.float32),
            pltpu.VMEM((NP, NA), jnp.float32),
            pltpu.SemaphoreType.DMA,
            pltpu.SemaphoreType.DMA,
            pltpu.SemaphoreType.DMA,
            pltpu.SemaphoreType.DMA,
            pltpu.SemaphoreType.DMA,
            pltpu.SemaphoreType.DMA,
        ],
    )
    return fn(scp, ei, ea, zero)


def _layernorm(x, g, b):
    mu = jnp.mean(x, axis=1, keepdims=True)
    xc = x - mu
    var = jnp.mean(xc * xc, axis=1, keepdims=True)
    return xc * lax.rsqrt(var + 1e-5) * g + b


def _lrelu(x):
    return jnp.where(x >= 0, x, 0.01 * x)


def _head_body(a_ref, vs_ref, x_ref, we_ref, w1a_ref, w1c_ref, w1b_ref,
               b1_ref, g1_ref, be1_ref, w2_ref, b2_ref, g2_ref, be2_ref,
               w3_ref, b3_ref, act_ref, reg_ref, *, n_real, total_nodes,
               n_steps):
    step = pl.program_id(0)
    rows = NP * G2
    amat = a_ref[...]
    vs = vs_ref[...].astype(jnp.float32)
    x = x_ref[...]
    w = we_ref[...]
    wpad = jnp.broadcast_to(w, (NA - NP, 256))
    colA = lax.broadcasted_iota(jnp.int32, (rows, NA), 1)
    colX = lax.broadcasted_iota(jnp.int32, (rows, 128), 1)
    msgs = []
    for i in range(G2):
        ai = amat[NP * i:NP * i + NP]
        vi = vs[NP * i:NP * i + NP, :256]
        vaug = jnp.concatenate([vi, wpad], axis=0)
        msgs.append(jnp.dot(ai, vaug, preferred_element_type=jnp.float32))
    msg = jnp.concatenate(msgs, axis=0)
    den = jnp.sum(jnp.where(colA < NP, amat, 0.0), axis=1, keepdims=True)
    out = msg / (den + 1e-16) + vs[:, 256:]
    h = jnp.maximum(out, 0.0)
    xc1 = jnp.where(colX == 1, x, 0.0)
    ta_cols = []
    for i in range(G2):
        ta_i = jnp.sum(xc1[NP * i:NP * i + NP])
        ta_cols.append(jnp.full((NP, 1), ta_i, jnp.float32))
    ta = jnp.concatenate(ta_cols, axis=0)
    o1 = (jnp.dot(h, w1a_ref[...], preferred_element_type=jnp.float32)
          + jnp.dot(x, w1c_ref[...], preferred_element_type=jnp.float32)
          + ta * w1b_ref[...] + b1_ref[...])
    x1 = _lrelu(_layernorm(o1, g1_ref[...], be1_ref[...]))
    o2 = jnp.dot(x1, w2_ref[...],
                 preferred_element_type=jnp.float32) + b2_ref[...]
    x2 = _lrelu(_layernorm(o2, g2_ref[...], be2_ref[...]))
    z = jnp.sum(x2 * w3_ref[...], axis=1, keepdims=True) + b3_ref[0, 0]
    conc = jnp.maximum(z, 0.0) + jnp.log(1.0 + jnp.exp(-jnp.abs(z)))
    rowm = (lax.broadcasted_iota(jnp.int32, (rows, 1), 0) % NP) < n_real
    concm = jnp.where(rowm, conc, 0.0)
    regs = jnp.sum(jnp.abs(concm))
    for i in range(G2):
        ci = concm[NP * i:NP * i + NP]
        ssum = jnp.sum(ci)
        act_ref[NP * i:NP * i + NP, :] = ci / (ssum + 1e-20)
    tot = jnp.where(step == 0, 0.0, reg_ref[...]) + regs
    reg_ref[...] = jnp.where(step == n_steps - 1, tot / total_nodes, tot)


def _proj_call(xf, wall, ball, we):
    n = xf.shape[0]
    rows = NP * G1
    grid = (n // rows,)
    return pl.pallas_call(
        _proj_body,
        grid=grid,
        in_specs=[
            pl.BlockSpec((rows, 128), lambda i: (i, 0)),
            pl.BlockSpec((128, 1024), lambda i: (0, 0)),
            pl.BlockSpec((1, 1024), lambda i: (0, 0)),
            pl.BlockSpec((1, 256), lambda i: (0, 0)),
        ],
        out_specs=[
            pl.BlockSpec((rows, 512), lambda i: (i, 0)),
            pl.BlockSpec((rows, NA), lambda i: (i, 0)),
        ],
        out_shape=[
            jax.ShapeDtypeStruct((n, 512), jnp.bfloat16),
            jax.ShapeDtypeStruct((n, NA), jnp.float32),
        ],
    )(xf, wall, ball, we)


def _head_call(af, vs, xf, we, w1a, w1c, w1b, b1, g1, be1, w2, b2, g2, be2,
               w3r, b3, n_real, total_nodes):
    n = af.shape[0]
    rows = NP * G2
    grid = (n // rows,)
    body = functools.partial(_head_body, n_real=n_real,
                             total_nodes=total_nodes, n_steps=n // rows)
    const = lambda i: (0, 0)
    return pl.pallas_call(
        body,
        grid=grid,
        in_specs=[
            pl.BlockSpec((rows, NA), lambda i: (i, 0)),
            pl.BlockSpec((rows, 512), lambda i: (i, 0)),
            pl.BlockSpec((rows, 128), lambda i: (i, 0)),
            pl.BlockSpec((1, 256), const),
            pl.BlockSpec((256, 256), const),
            pl.BlockSpec((128, 256), const),
            pl.BlockSpec((1, 256), const),
            pl.BlockSpec((1, 256), const),
            pl.BlockSpec((1, 256), const),
            pl.BlockSpec((1, 256), const),
            pl.BlockSpec((256, 256), const),
            pl.BlockSpec((1, 256), const),
            pl.BlockSpec((1, 256), const),
            pl.BlockSpec((1, 256), const),
            pl.BlockSpec((1, 256), const),
            pl.BlockSpec((1, 1), const),
        ],
        out_specs=[
            pl.BlockSpec((rows, 1), lambda i: (i, 0)),
            pl.BlockSpec((1, 1), const),
        ],
        out_shape=[
            jax.ShapeDtypeStruct((n, 1), jnp.float32),
            jax.ShapeDtypeStruct((1, 1), jnp.float32),
        ],
    )(af, vs, xf, we, w1a, w1c, w1b, b1, g1, be1, w2, b2, g2, be2, w3r, b3)


def kernel(state, edge_index, edge_attr, Wq, bq, Wk, bk, Wv, bv, We, Wskip,
           bskip, W1, b1, g1, beta1, W2, b2, g2, beta2, W3, b3):
    B, Npg, Fin = state.shape
    x = jnp.concatenate(
        [state, jnp.broadcast_to(_POS[None], (B, Npg, 6))], axis=-1)
    xf = jnp.pad(x, ((0, 0), (0, NP - Npg), (0, 0))).reshape(B * NP, -1)

    wall = jnp.concatenate([Wq, Wk, Wv, Wskip], axis=1)
    ball = jnp.concatenate([bq, bk, bv, bskip])[None]
    vs, scp = _proj_call(xf, wall, ball, We)

    zero = jnp.zeros((NP, NA), jnp.float32)
    amat = _sc_edge(scp, edge_index.astype(jnp.int32),
                    edge_attr.reshape(-1), zero)

    act, reg = _head_call(
        amat, vs, xf, We,
        W1[:256], W1[257:], W1[256:257], b1[None], g1[None], beta1[None],
        W2, b2[None], g2[None], beta2[None], W3.T, b3[None],
        Npg, B * Npg)
    action = act.reshape(B, NP)[:, :Npg]
    return (action, reg[0, 0])


# speedup vs baseline: 1.1052x; 1.0552x over previous
import functools

import jax
import jax.numpy as jnp
import numpy as np
from jax import lax
from jax.experimental import pallas as pl
from jax.experimental.pallas import tpu as pltpu
from jax.experimental.pallas import tpu_sc as plsc

_POS_INDICES = [120, 124, 128, 132, 136, 140, 144, 148, 152, 237, 241, 245,
                249, 253, 257, 261, 265, 269, 354, 358, 362, 366, 370, 374,
                378, 382, 386, 471, 475, 479, 483, 487, 491, 495, 499, 503,
                588, 592, 596, 600, 604, 608, 612, 616, 620, 705, 709, 713,
                717, 721, 725, 729, 733, 737, 822, 826, 830, 834, 838, 842,
                846, 850, 854, 48, 53, 60, 67, 73, 157, 352, 388, 583, 586,
                817, 901, 906, 913, 920, 926]


def _positions():
    width, height = 39, 25
    pf = np.zeros((79, 6), dtype=np.float32)
    for i, p in enumerate(_POS_INDICES):
        x = p % width
        y = p // width
        xn = x / (width - 1)
        yn = y / (height - 1)
        pf[i, 0] = xn
        pf[i, 1] = yn
        pf[i, 2] = (np.sin(xn * 2 * np.pi) + 1) / 2
        pf[i, 3] = (np.cos(xn * 2 * np.pi) + 1) / 2
        pf[i, 4] = (np.sin(yn * 2 * np.pi) + 1) / 2
        pf[i, 5] = (np.cos(yn * 2 * np.pi) + 1) / 2
    return pf


_POS = _positions()

NP = 80
NA = 128
TCOL = 96
G1 = 8
G2 = 8
_INV_SQRT_D = 1.0 / 16.0


def _proj_body(x_ref, w_ref, b_ref, we_ref, vs_ref, sc_ref):
    x = x_ref[...]
    y = jnp.dot(x, w_ref[...], preferred_element_type=jnp.float32) + b_ref[...]
    q = y[:, :256]
    k = y[:, 256:512]
    vs_ref[...] = y[:, 512:].astype(jnp.bfloat16)
    w = we_ref[...]
    t = jnp.sum(q * w, axis=1, keepdims=True) * _INV_SQRT_D
    zpad = jnp.zeros((NA - NP, 256), jnp.float32)
    colid = lax.broadcasted_iota(jnp.int32, (NP, NA), 1)
    for i in range(G1):
        qi = q[NP * i:NP * i + NP]
        ki = k[NP * i:NP * i + NP]
        kaug = jnp.concatenate([ki, zpad], axis=0)
        s = lax.dot_general(qi, kaug, (((1,), (1,)), ((), ())),
                            preferred_element_type=jnp.float32) * _INV_SQRT_D
        ti = t[NP * i:NP * i + NP]
        c = jnp.max(s, axis=1, keepdims=True) + jnp.abs(ti)
        sc_ref[NP * i:NP * i + NP, :] = jnp.where(colid == TCOL, ti, s - c)


def _sc_edge_body(sc_hbm, ei_hbm, ea_hbm, zero_hbm, out_hbm,
                  pidx_v, dst_v, a_v, sc_v0, sc_v1, acc_v0, acc_v1,
                  sem_sc0, sem_sc1, sem_z0, sem_z1, sem_o0, sem_o1,
                  n_graphs, n_groups):
    cid = lax.axis_index("c")
    sid = lax.axis_index("s")
    wid = sid * 2 + cid
    gpw = n_graphs // 32
    pltpu.sync_copy(ei_hbm.at[0], pidx_v)
    pltpu.sync_copy(ei_hbm.at[1], dst_v)
    pltpu.sync_copy(ea_hbm, a_v)
    lane80 = 80 + lax.iota(jnp.int32, 16)
    tcol = jnp.full((16,), TCOL, jnp.int32)

    def pack_body(j):
        slc = pl.ds(j * 16, 16)
        pidx_v[slc] = dst_v[slc] * NA + pidx_v[slc]

    plsc.parallel_loop(0, n_groups, unroll=8)(pack_body)

    bufs = [(sc_v0, acc_v0, sem_sc0, sem_z0, sem_o0),
            (sc_v1, acc_v1, sem_sc1, sem_z1, sem_o1)]

    def make_edge(sc_v, acc_v):
        def edge_body(j):
            slc = pl.ds(j * 16, 16)
            pi = pidx_v[slc]
            ai = a_v[slc]
            di = lax.shift_right_logical(pi, 7)
            si = jnp.bitwise_and(pi, NA - 1)
            base = plsc.load_gather(sc_v, [di, si])
            tv = plsc.load_gather(sc_v, [di, tcol])
            e = jnp.exp(base + ai * tv)
            plsc.addupdate_scatter(acc_v, [di, si], e)
            plsc.addupdate_scatter(acc_v, [di, lane80], e * ai)
        return edge_body

    g0 = wid * gpw
    in_h = {}
    z_h = {}
    out_h = {}
    in_h[0] = pltpu.async_copy(sc_hbm.at[pl.ds(g0 * NP, NP)],
                               bufs[0][0], bufs[0][2])
    z_h[0] = pltpu.async_copy(zero_hbm, bufs[0][1], bufs[0][3])
    for gi in range(gpw):
        sc_v, acc_v, s_sc, s_z, s_o = bufs[gi % 2]
        in_h.pop(gi).wait()
        z_h.pop(gi).wait()
        if gi + 1 < gpw:
            nb = bufs[(gi + 1) % 2]
            if gi >= 1:
                out_h.pop(gi - 1).wait()
            in_h[gi + 1] = pltpu.async_copy(
                sc_hbm.at[pl.ds((g0 + gi + 1) * NP, NP)], nb[0], nb[2])
            z_h[gi + 1] = pltpu.async_copy(zero_hbm, nb[1], nb[3])
        plsc.parallel_loop(0, n_groups, unroll=8)(make_edge(sc_v, acc_v))
        out_h[gi] = pltpu.async_copy(
            acc_v, out_hbm.at[pl.ds((g0 + gi) * NP, NP)], s_o)
    for h in out_h.values():
        h.wait()


def _sc_edge(scp, ei, ea, zero):
    n = scp.shape[0]
    e = ei.shape[1]
    mesh = plsc.VectorSubcoreMesh(core_axis_name="c", subcore_axis_name="s")
    fn = pl.kernel(
        functools.partial(_sc_edge_body, n_graphs=n // NP, n_groups=e // 16),
        mesh=mesh,
        compiler_params=pltpu.CompilerParams(needs_layout_passes=False),
        out_type=jax.ShapeDtypeStruct((n, NA), jnp.float32),
        scratch_types=[
            pltpu.VMEM((e,), jnp.int32),
            pltpu.VMEM((e,), jnp.int32),
            pltpu.VMEM((e,), jnp.float32),
            pltpu.VMEM((NP, NA), jnp.float32),
            pltpu.VMEM((NP, NA), jnp.float32),
            pltpu.VMEM((NP, NA), jnp.float32),
            pltpu.VMEM((NP, NA), jnp.float32),
            pltpu.SemaphoreType.DMA,
            pltpu.SemaphoreType.DMA,
            pltpu.SemaphoreType.DMA,
            pltpu.SemaphoreType.DMA,
            pltpu.SemaphoreType.DMA,
            pltpu.SemaphoreType.DMA,
        ],
    )
    return fn(scp, ei, ea, zero)


def _layernorm(x, g, b):
    mu = jnp.mean(x, axis=1, keepdims=True)
    xc = x - mu
    var = jnp.mean(xc * xc, axis=1, keepdims=True)
    return xc * lax.rsqrt(var + 1e-5) * g + b


def _lrelu(x):
    return jnp.where(x >= 0, x, 0.01 * x)


def _head_body(a_ref, vs_ref, x_ref, we_ref, w1a_ref, w1c_ref, w1b_ref,
               b1_ref, g1_ref, be1_ref, w2_ref, b2_ref, g2_ref, be2_ref,
               w3_ref, b3_ref, act_ref, reg_ref, *, n_real, total_nodes,
               n_steps):
    step = pl.program_id(0)
    rows = NP * G2
    amat = a_ref[...]
    vs = vs_ref[...].astype(jnp.float32)
    x = x_ref[...]
    w = we_ref[...]
    wpad = jnp.broadcast_to(w, (NA - NP, 256))
    colA = lax.broadcasted_iota(jnp.int32, (rows, NA), 1)
    colX = lax.broadcasted_iota(jnp.int32, (rows, 128), 1)
    msgs = []
    for i in range(G2):
        ai = amat[NP * i:NP * i + NP]
        vi = vs[NP * i:NP * i + NP, :256]
        vaug = jnp.concatenate([vi, wpad], axis=0)
        msgs.append(jnp.dot(ai, vaug, preferred_element_type=jnp.float32))
    msg = jnp.concatenate(msgs, axis=0)
    den = jnp.sum(jnp.where(colA < NP, amat, 0.0), axis=1, keepdims=True)
    out = msg / (den + 1e-16) + vs[:, 256:]
    h = jnp.maximum(out, 0.0)
    xc1 = jnp.where(colX == 1, x, 0.0)
    ta_cols = []
    for i in range(G2):
        ta_i = jnp.sum(xc1[NP * i:NP * i + NP])
        ta_cols.append(jnp.full((NP, 1), ta_i, jnp.float32))
    ta = jnp.concatenate(ta_cols, axis=0)
    o1 = (jnp.dot(h, w1a_ref[...], preferred_element_type=jnp.float32)
          + jnp.dot(x, w1c_ref[...], preferred_element_type=jnp.float32)
          + ta * w1b_ref[...] + b1_ref[...])
    x1 = _lrelu(_layernorm(o1, g1_ref[...], be1_ref[...]))
    o2 = jnp.dot(x1, w2_ref[...],
                 preferred_element_type=jnp.float32) + b2_ref[...]
    x2 = _lrelu(_layernorm(o2, g2_ref[...], be2_ref[...]))
    z = jnp.sum(x2 * w3_ref[...], axis=1, keepdims=True) + b3_ref[0, 0]
    conc = jnp.maximum(z, 0.0) + jnp.log(1.0 + jnp.exp(-jnp.abs(z)))
    rowm = (lax.broadcasted_iota(jnp.int32, (rows, 1), 0) % NP) < n_real
    concm = jnp.where(rowm, conc, 0.0)
    regs = jnp.sum(jnp.abs(concm))
    for i in range(G2):
        ci = concm[NP * i:NP * i + NP]
        ssum = jnp.sum(ci)
        act_ref[NP * i:NP * i + NP, :] = ci / (ssum + 1e-20)
    tot = jnp.where(step == 0, 0.0, reg_ref[...]) + regs
    reg_ref[...] = jnp.where(step == n_steps - 1, tot / total_nodes, tot)


def _proj_call(xf, wall, ball, we):
    n = xf.shape[0]
    rows = NP * G1
    grid = (n // rows,)
    return pl.pallas_call(
        _proj_body,
        grid=grid,
        in_specs=[
            pl.BlockSpec((rows, 128), lambda i: (i, 0)),
            pl.BlockSpec((128, 1024), lambda i: (0, 0)),
            pl.BlockSpec((1, 1024), lambda i: (0, 0)),
            pl.BlockSpec((1, 256), lambda i: (0, 0)),
        ],
        out_specs=[
            pl.BlockSpec((rows, 512), lambda i: (i, 0)),
            pl.BlockSpec((rows, NA), lambda i: (i, 0)),
        ],
        out_shape=[
            jax.ShapeDtypeStruct((n, 512), jnp.bfloat16),
            jax.ShapeDtypeStruct((n, NA), jnp.float32),
        ],
    )(xf, wall, ball, we)


def _head_call(af, vs, xf, we, w1a, w1c, w1b, b1, g1, be1, w2, b2, g2, be2,
               w3r, b3, n_real, total_nodes):
    n = af.shape[0]
    rows = NP * G2
    grid = (n // rows,)
    body = functools.partial(_head_body, n_real=n_real,
                             total_nodes=total_nodes, n_steps=n // rows)
    const = lambda i: (0, 0)
    return pl.pallas_call(
        body,
        grid=grid,
        in_specs=[
            pl.BlockSpec((rows, NA), lambda i: (i, 0)),
            pl.BlockSpec((rows, 512), lambda i: (i, 0)),
            pl.BlockSpec((rows, 128), lambda i: (i, 0)),
            pl.BlockSpec((1, 256), const),
            pl.BlockSpec((256, 256), const),
            pl.BlockSpec((128, 256), const),
            pl.BlockSpec((1, 256), const),
            pl.BlockSpec((1, 256), const),
            pl.BlockSpec((1, 256), const),
            pl.BlockSpec((1, 256), const),
            pl.BlockSpec((256, 256), const),
            pl.BlockSpec((1, 256), const),
            pl.BlockSpec((1, 256), const),
            pl.BlockSpec((1, 256), const),
            pl.BlockSpec((1, 256), const),
            pl.BlockSpec((1, 1), const),
        ],
        out_specs=[
            pl.BlockSpec((rows, 1), lambda i: (i, 0)),
            pl.BlockSpec((1, 1), const),
        ],
        out_shape=[
            jax.ShapeDtypeStruct((n, 1), jnp.float32),
            jax.ShapeDtypeStruct((1, 1), jnp.float32),
        ],
    )(af, vs, xf, we, w1a, w1c, w1b, b1, g1, be1, w2, b2, g2, be2, w3r, b3)


def kernel(state, edge_index, edge_attr, Wq, bq, Wk, bk, Wv, bv, We, Wskip,
           bskip, W1, b1, g1, beta1, W2, b2, g2, beta2, W3, b3):
    B, Npg, Fin = state.shape
    wall = jnp.concatenate([Wq, Wk, Wv, Wskip], axis=1)
    ball = jnp.concatenate([bq, bk, bv, bskip])[None]
    ei = edge_index.astype(jnp.int32)
    ea = edge_attr.reshape(-1)
    zero = jnp.zeros((NP, NA), jnp.float32)

    hb = B // 2
    halves = []
    for h in range(2):
        sh = state[h * hb:(h + 1) * hb]
        x = jnp.concatenate(
            [sh, jnp.broadcast_to(_POS[None], (hb, Npg, 6))], axis=-1)
        xf = jnp.pad(x, ((0, 0), (0, NP - Npg), (0, 0))).reshape(hb * NP, -1)
        vs, scp = _proj_call(xf, wall, ball, We)
        halves.append((xf, vs, scp))

    amats = [_sc_edge(scp, ei, ea, zero) for (_, _, scp) in halves]

    acts = []
    reg = jnp.float32(0.0)
    for (xf, vs, _), amat in zip(halves, amats):
        act, r = _head_call(
            amat, vs, xf, We,
            W1[:256], W1[257:], W1[256:257], b1[None], g1[None], beta1[None],
            W2, b2[None], g2[None], beta2[None], W3.T, b3[None],
            Npg, B * Npg)
        acts.append(act.reshape(hb, NP)[:, :Npg])
        reg = reg + r[0, 0]
    action = jnp.concatenate(acts, axis=0)
    return (action, reg)


# speedup vs baseline: 1.2303x; 1.1132x over previous
import functools

import jax
import jax.numpy as jnp
import numpy as np
from jax import lax
from jax.experimental import pallas as pl
from jax.experimental.pallas import tpu as pltpu
from jax.experimental.pallas import tpu_sc as plsc

_POS_INDICES = [120, 124, 128, 132, 136, 140, 144, 148, 152, 237, 241, 245,
                249, 253, 257, 261, 265, 269, 354, 358, 362, 366, 370, 374,
                378, 382, 386, 471, 475, 479, 483, 487, 491, 495, 499, 503,
                588, 592, 596, 600, 604, 608, 612, 616, 620, 705, 709, 713,
                717, 721, 725, 729, 733, 737, 822, 826, 830, 834, 838, 842,
                846, 850, 854, 48, 53, 60, 67, 73, 157, 352, 388, 583, 586,
                817, 901, 906, 913, 920, 926]


def _positions():
    width, height = 39, 25
    pf = np.zeros((79, 6), dtype=np.float32)
    for i, p in enumerate(_POS_INDICES):
        x = p % width
        y = p // width
        xn = x / (width - 1)
        yn = y / (height - 1)
        pf[i, 0] = xn
        pf[i, 1] = yn
        pf[i, 2] = (np.sin(xn * 2 * np.pi) + 1) / 2
        pf[i, 3] = (np.cos(xn * 2 * np.pi) + 1) / 2
        pf[i, 4] = (np.sin(yn * 2 * np.pi) + 1) / 2
        pf[i, 5] = (np.cos(yn * 2 * np.pi) + 1) / 2
    return pf


_POS = _positions()

NP = 80
NA = 128
NS = 96
TCOL = 80
G1 = 16
G2 = 16
_INV_SQRT_D = 1.0 / 16.0


def _proj_body(x_ref, w_ref, b_ref, we_ref, vs_ref, sc_ref):
    x = x_ref[...]
    y = jnp.dot(x, w_ref[...], preferred_element_type=jnp.float32) + b_ref[...]
    q = y[:, :256]
    k = y[:, 256:512]
    vs_ref[...] = y[:, 512:].astype(jnp.bfloat16)
    w = we_ref[...]
    t = jnp.sum(q * w, axis=1, keepdims=True) * _INV_SQRT_D
    zpad = jnp.zeros((NS - NP, 256), jnp.float32)
    colid = lax.broadcasted_iota(jnp.int32, (NP, NS), 1)
    for i in range(G1):
        qi = q[NP * i:NP * i + NP]
        ki = k[NP * i:NP * i + NP]
        kaug = jnp.concatenate([ki, zpad], axis=0)
        s = lax.dot_general(qi, kaug, (((1,), (1,)), ((), ())),
                            preferred_element_type=jnp.float32) * _INV_SQRT_D
        ti = t[NP * i:NP * i + NP]
        c = jnp.max(s, axis=1, keepdims=True) + jnp.abs(ti)
        sc_ref[NP * i:NP * i + NP, :NS] = jnp.where(colid == TCOL, ti, s - c)


def _sc_edge_body(sc_hbm, ei_hbm, ea_hbm, zero_hbm, out_hbm,
                  pidx_v, dst_v, a_v, sc_v0, sc_v1, acc_v0, acc_v1,
                  sem_sc0, sem_sc1, sem_z0, sem_z1, sem_o0, sem_o1,
                  n_graphs, n_groups):
    cid = lax.axis_index("c")
    sid = lax.axis_index("s")
    wid = sid * 2 + cid
    gpw = n_graphs // 32
    pltpu.sync_copy(ei_hbm.at[0], pidx_v)
    pltpu.sync_copy(ei_hbm.at[1], dst_v)
    pltpu.sync_copy(ea_hbm, a_v)
    lane80 = 80 + lax.iota(jnp.int32, 16)
    tcol = jnp.full((16,), TCOL, jnp.int32)

    def pack_body(j):
        slc = pl.ds(j * 16, 16)
        pidx_v[slc] = dst_v[slc] * NA + pidx_v[slc]

    plsc.parallel_loop(0, n_groups, unroll=8)(pack_body)

    bufs = [(sc_v0, acc_v0, sem_sc0, sem_z0, sem_o0),
            (sc_v1, acc_v1, sem_sc1, sem_z1, sem_o1)]

    def make_edge(sc_v, acc_v):
        def edge_body(j):
            slc = pl.ds(j * 16, 16)
            pi = pidx_v[slc]
            ai = a_v[slc]
            di = lax.shift_right_logical(pi, 7)
            si = jnp.bitwise_and(pi, NA - 1)
            base = plsc.load_gather(sc_v, [di, si])
            tv = plsc.load_gather(sc_v, [di, tcol])
            e = jnp.exp(base + ai * tv)
            plsc.addupdate_scatter(acc_v, [di, si], e)
            plsc.addupdate_scatter(acc_v, [di, lane80], e * ai)
        return edge_body

    g0 = wid * gpw
    in_h = {}
    z_h = {}
    out_h = {}
    in_h[0] = pltpu.async_copy(sc_hbm.at[pl.ds(g0 * NP, NP)],
                               bufs[0][0], bufs[0][2])
    z_h[0] = pltpu.async_copy(zero_hbm, bufs[0][1], bufs[0][3])
    for gi in range(gpw):
        sc_v, acc_v, s_sc, s_z, s_o = bufs[gi % 2]
        in_h.pop(gi).wait()
        z_h.pop(gi).wait()
        if gi + 1 < gpw:
            nb = bufs[(gi + 1) % 2]
            if gi >= 1:
                out_h.pop(gi - 1).wait()
            in_h[gi + 1] = pltpu.async_copy(
                sc_hbm.at[pl.ds((g0 + gi + 1) * NP, NP)], nb[0], nb[2])
            z_h[gi + 1] = pltpu.async_copy(zero_hbm, nb[1], nb[3])
        plsc.parallel_loop(0, n_groups, unroll=8)(make_edge(sc_v, acc_v))
        out_h[gi] = pltpu.async_copy(
            acc_v, out_hbm.at[pl.ds((g0 + gi) * NP, NP)], s_o)
    for h in out_h.values():
        h.wait()


def _sc_edge(scp, ei, ea, zero):
    n = scp.shape[0]
    e = ei.shape[1]
    mesh = plsc.VectorSubcoreMesh(core_axis_name="c", subcore_axis_name="s")
    fn = pl.kernel(
        functools.partial(_sc_edge_body, n_graphs=n // NP, n_groups=e // 16),
        mesh=mesh,
        compiler_params=pltpu.CompilerParams(needs_layout_passes=False),
        out_type=jax.ShapeDtypeStruct((n, NA), jnp.float32),
        scratch_types=[
            pltpu.VMEM((e,), jnp.int32),
            pltpu.VMEM((e,), jnp.int32),
            pltpu.VMEM((e,), jnp.float32),
            pltpu.VMEM((NP, NA), jnp.float32),
            pltpu.VMEM((NP, NA), jnp.float32),
            pltpu.VMEM((NP, NA), jnp.float32),
            pltpu.VMEM((NP, NA), jnp.float32),
            pltpu.SemaphoreType.DMA,
            pltpu.SemaphoreType.DMA,
            pltpu.SemaphoreType.DMA,
            pltpu.SemaphoreType.DMA,
            pltpu.SemaphoreType.DMA,
            pltpu.SemaphoreType.DMA,
        ],
    )
    return fn(scp, ei, ea, zero)


def _layernorm(x, g, b):
    mu = jnp.mean(x, axis=1, keepdims=True)
    xc = x - mu
    var = jnp.mean(xc * xc, axis=1, keepdims=True)
    return xc * lax.rsqrt(var + 1e-5) * g + b


def _lrelu(x):
    return jnp.where(x >= 0, x, 0.01 * x)


def _head_body(a_ref, vs_ref, x_ref, we_ref, w1a_ref, w1c_ref, w1b_ref,
               b1_ref, g1_ref, be1_ref, w2_ref, b2_ref, g2_ref, be2_ref,
               w3_ref, b3_ref, act_ref, reg_ref, *, n_real, total_nodes,
               n_steps):
    step = pl.program_id(0)
    rows = NP * G2
    amat = a_ref[...]
    vs = vs_ref[...].astype(jnp.float32)
    x = x_ref[...]
    w = we_ref[...]
    wpad = jnp.broadcast_to(w, (NA - NP, 256))
    colA = lax.broadcasted_iota(jnp.int32, (rows, NA), 1)
    colX = lax.broadcasted_iota(jnp.int32, (rows, 128), 1)
    msgs = []
    for i in range(G2):
        ai = amat[NP * i:NP * i + NP]
        vi = vs[NP * i:NP * i + NP, :256]
        vaug = jnp.concatenate([vi, wpad], axis=0)
        msgs.append(jnp.dot(ai, vaug, preferred_element_type=jnp.float32))
    msg = jnp.concatenate(msgs, axis=0)
    den = jnp.sum(jnp.where(colA < NP, amat, 0.0), axis=1, keepdims=True)
    out = msg / (den + 1e-16) + vs[:, 256:]
    h = jnp.maximum(out, 0.0)
    xc1 = jnp.where(colX == 1, x, 0.0)
    ta_cols = []
    for i in range(G2):
        ta_i = jnp.sum(xc1[NP * i:NP * i + NP])
        ta_cols.append(jnp.full((NP, 1), ta_i, jnp.float32))
    ta = jnp.concatenate(ta_cols, axis=0)
    o1 = (jnp.dot(h, w1a_ref[...], preferred_element_type=jnp.float32)
          + jnp.dot(x, w1c_ref[...], preferred_element_type=jnp.float32)
          + ta * w1b_ref[...] + b1_ref[...])
    x1 = _lrelu(_layernorm(o1, g1_ref[...], be1_ref[...]))
    o2 = jnp.dot(x1, w2_ref[...],
                 preferred_element_type=jnp.float32) + b2_ref[...]
    x2 = _lrelu(_layernorm(o2, g2_ref[...], be2_ref[...]))
    z = jnp.sum(x2 * w3_ref[...], axis=1, keepdims=True) + b3_ref[0, 0]
    conc = jnp.maximum(z, 0.0) + jnp.log(1.0 + jnp.exp(-jnp.abs(z)))
    rowm = (lax.broadcasted_iota(jnp.int32, (rows, 1), 0) % NP) < n_real
    concm = jnp.where(rowm, conc, 0.0)
    regs = jnp.sum(jnp.abs(concm))
    for i in range(G2):
        ci = concm[NP * i:NP * i + NP]
        ssum = jnp.sum(ci)
        act_ref[NP * i:NP * i + NP, :] = ci / (ssum + 1e-20)
    tot = jnp.where(step == 0, 0.0, reg_ref[...]) + regs
    reg_ref[...] = jnp.where(step == n_steps - 1, tot / total_nodes, tot)


def _proj_call(xf, wall, ball, we):
    n = xf.shape[0]
    rows = NP * G1
    grid = (n // rows,)
    return pl.pallas_call(
        _proj_body,
        grid=grid,
        in_specs=[
            pl.BlockSpec((rows, 128), lambda i: (i, 0)),
            pl.BlockSpec((128, 1024), lambda i: (0, 0)),
            pl.BlockSpec((1, 1024), lambda i: (0, 0)),
            pl.BlockSpec((1, 256), lambda i: (0, 0)),
        ],
        out_specs=[
            pl.BlockSpec((rows, 512), lambda i: (i, 0)),
            pl.BlockSpec((rows, NA), lambda i: (i, 0)),
        ],
        out_shape=[
            jax.ShapeDtypeStruct((n, 512), jnp.bfloat16),
            jax.ShapeDtypeStruct((n, NA), jnp.float32),
        ],
    )(xf, wall, ball, we)


def _head_call(af, vs, xf, we, w1a, w1c, w1b, b1, g1, be1, w2, b2, g2, be2,
               w3r, b3, n_real, total_nodes):
    n = af.shape[0]
    rows = NP * G2
    grid = (n // rows,)
    body = functools.partial(_head_body, n_real=n_real,
                             total_nodes=total_nodes, n_steps=n // rows)
    const = lambda i: (0, 0)
    return pl.pallas_call(
        body,
        grid=grid,
        in_specs=[
            pl.BlockSpec((rows, NA), lambda i: (i, 0)),
            pl.BlockSpec((rows, 512), lambda i: (i, 0)),
            pl.BlockSpec((rows, 128), lambda i: (i, 0)),
            pl.BlockSpec((1, 256), const),
            pl.BlockSpec((256, 256), const),
            pl.BlockSpec((128, 256), const),
            pl.BlockSpec((1, 256), const),
            pl.BlockSpec((1, 256), const),
            pl.BlockSpec((1, 256), const),
            pl.BlockSpec((1, 256), const),
            pl.BlockSpec((256, 256), const),
            pl.BlockSpec((1, 256), const),
            pl.BlockSpec((1, 256), const),
            pl.BlockSpec((1, 256), const),
            pl.BlockSpec((1, 256), const),
            pl.BlockSpec((1, 1), const),
        ],
        out_specs=[
            pl.BlockSpec((rows, 1), lambda i: (i, 0)),
            pl.BlockSpec((1, 1), const),
        ],
        out_shape=[
            jax.ShapeDtypeStruct((n, 1), jnp.float32),
            jax.ShapeDtypeStruct((1, 1), jnp.float32),
        ],
    )(af, vs, xf, we, w1a, w1c, w1b, b1, g1, be1, w2, b2, g2, be2, w3r, b3)


def kernel(state, edge_index, edge_attr, Wq, bq, Wk, bk, Wv, bv, We, Wskip,
           bskip, W1, b1, g1, beta1, W2, b2, g2, beta2, W3, b3):
    B, Npg, Fin = state.shape
    wall = jnp.concatenate([Wq, Wk, Wv, Wskip], axis=1)
    ball = jnp.concatenate([bq, bk, bv, bskip])[None]
    ei = edge_index.astype(jnp.int32)
    ea = edge_attr.reshape(-1)
    zero = jnp.zeros((NP, NA), jnp.float32)

    hb = B // 2
    halves = []
    for h in range(2):
        sh = state[h * hb:(h + 1) * hb]
        x = jnp.concatenate(
            [sh, jnp.broadcast_to(_POS[None], (hb, Npg, 6))], axis=-1)
        xf = jnp.pad(x, ((0, 0), (0, NP - Npg), (0, 0))).reshape(hb * NP, -1)
        vs, scp = _proj_call(xf, wall, ball, We)
        halves.append((xf, vs, scp))

    amats = [_sc_edge(scp, ei, ea, zero) for (_, _, scp) in halves]

    acts = []
    reg = jnp.float32(0.0)
    for (xf, vs, _), amat in zip(halves, amats):
        act, r = _head_call(
            amat, vs, xf, We,
            W1[:256], W1[257:], W1[256:257], b1[None], g1[None], beta1[None],
            W2, b2[None], g2[None], beta2[None], W3.T, b3[None],
            Npg, B * Npg)
        acts.append(act.reshape(hb, NP)[:, :Npg])
        reg = reg + r[0, 0]
    action = jnp.concatenate(acts, axis=0)
    return (action, reg)
